# Initial kernel scaffold; baseline (speedup 1.0000x reference)
import functools

import jax
import jax.numpy as jnp
from jax import lax
from jax.experimental import pallas as pl
from jax.experimental.pallas import tpu as pltpu
from jax.experimental.pallas import tpu_sc as plsc

_NC = 2
_NS = 16
_L = 16
_C = 128


def _sc_aggregate(hs, ad, srcp, dstp, zeros, *, n_pad, k_chunks, dh, hdim):
    w = dh + 16
    groups = dh // _L
    rows_per_tile = n_pad // _NS

    mesh = plsc.VectorSubcoreMesh(core_axis_name="c", subcore_axis_name="s")

    @functools.partial(
        pl.kernel,
        out_type=jax.ShapeDtypeStruct((_NC, n_pad, w), jnp.float32),
        mesh=mesh,
        scratch_types=[
            pltpu.VMEM((_C,), jnp.int32),
            pltpu.VMEM((_C,), jnp.int32),
            pltpu.VMEM((_C, w), jnp.float32),
            pltpu.VMEM((_C, 16), jnp.float32),
            pltpu.VMEM_SHARED((n_pad, w), jnp.float32),
            pltpu.SemaphoreType.DMA,
        ],
    )
    def body(hs_hbm, ad_hbm, src_hbm, dst_hbm, z_hbm, out_hbm,
             src_v, dst_v, msg_v, ad_v, acc, sem):
        cid = lax.axis_index("c")
        sid = lax.axis_index("s")
        wid = sid * _NC + cid
        r0 = sid * rows_per_tile
        pltpu.sync_copy(z_hbm.at[pl.ds(r0, rows_per_tile)],
                        acc.at[pl.ds(r0, rows_per_tile)])
        plsc.subcore_barrier()
        ebase = wid * (k_chunks * _C)
        iota = lax.iota(jnp.int32, _L)

        def chunk(j, carry):
            off = ebase + j * _C
            pltpu.sync_copy(src_hbm.at[pl.ds(off, _C)], src_v)
            pltpu.sync_copy(dst_hbm.at[pl.ds(off, _C)], dst_v)
            pltpu.async_copy(hs_hbm.at[src_v], msg_v, sem).wait()
            pltpu.async_copy(ad_hbm.at[dst_v], ad_v, sem).wait()

            def edge(e, c2):
                vas = msg_v[e, pl.ds(dh, _L)]
                vad = ad_v[e, :]
                z = vas + vad
                z = jnp.where(z >= 0, z, 0.2 * z)
                msg_v[e, pl.ds(dh, _L)] = jnp.exp(z)
                row = jnp.full((_L,), e, jnp.int32)
                for g in range(groups):
                    wcol = dh + (g * _L + iota) // hdim
                    wv = plsc.load_gather(msg_v, [row, wcol])
                    msg_v[e, pl.ds(g * _L, _L)] = (
                        msg_v[e, pl.ds(g * _L, _L)] * wv)
                return c2

            lax.fori_loop(0, _C, edge, 0)
            pltpu.sync_copy(msg_v, acc.at[dst_v], add=True)
            return carry

        lax.fori_loop(0, k_chunks, chunk, 0)
        plsc.subcore_barrier()
        pltpu.sync_copy(acc.at[pl.ds(r0, rows_per_tile)],
                        out_hbm.at[cid, pl.ds(r0, rows_per_tile)])

    return body(hs, ad, srcp, dstp, zeros)


def _tc_layer1(x, w1, a1s, a1d, rb):
    n, d_in = x.shape
    hid = w1.shape[1]
    heads = a1s.shape[1]

    def body(x_ref, w1_ref, a1s_ref, a1d_ref, hs_ref, ad_ref):
        h = jnp.dot(x_ref[...], w1_ref[...], preferred_element_type=jnp.float32)
        als = jnp.dot(h, a1s_ref[...], preferred_element_type=jnp.float32)
        ald = jnp.dot(h, a1d_ref[...], preferred_element_type=jnp.float32)
        pad = jnp.zeros((rb, 16 - heads), jnp.float32)
        hs_ref[...] = jnp.concatenate([h, als, pad], axis=1)
        ad_ref[...] = jnp.concatenate([ald, pad], axis=1)

    return pl.pallas_call(
        body,
        grid=(n // rb,),
        in_specs=[
            pl.BlockSpec((rb, d_in), lambda i: (i, 0)),
            pl.BlockSpec((d_in, hid), lambda i: (0, 0)),
            pl.BlockSpec((hid, heads), lambda i: (0, 0)),
            pl.BlockSpec((hid, heads), lambda i: (0, 0)),
        ],
        out_specs=[
            pl.BlockSpec((rb, hid + 16), lambda i: (i, 0)),
            pl.BlockSpec((rb, 16), lambda i: (i, 0)),
        ],
        out_shape=[
            jax.ShapeDtypeStruct((n, hid + 16), jnp.float32),
            jax.ShapeDtypeStruct((n, 16), jnp.float32),
        ],
    )(x, w1, a1s, a1d)


def _tc_layer2(p0, p1, bexp, b1, w2, a2v, rb):
    n = p0.shape[0]
    hid1 = b1.shape[1]
    d_out = w2.shape[1]

    def body(p0_ref, p1_ref, bexp_ref, b1_ref, w2_ref, a2v_ref,
             hs2_ref, ad2_ref):
        s = p0_ref[...] + p1_ref[...]
        num = s[:, :hid1]
        den = s[:, hid1:hid1 + 8]
        r = 1.0 / (den + 1e-16)
        rexp = jnp.dot(r, bexp_ref[...], preferred_element_type=jnp.float32)
        h1 = num * rexp + b1_ref[...]
        h1 = jnp.where(h1 > 0, h1, jnp.expm1(h1))
        h2 = jnp.dot(h1, w2_ref[...], preferred_element_type=jnp.float32)
        al2 = jnp.dot(h2, a2v_ref[...], preferred_element_type=jnp.float32)
        pad = jnp.zeros((rb, 15), jnp.float32)
        hs2_ref[...] = jnp.concatenate([h2, al2[:, :1], pad], axis=1)
        ad2_ref[...] = jnp.concatenate([al2[:, 1:2], pad], axis=1)

    wp = p0.shape[1]
    return pl.pallas_call(
        body,
        grid=(n // rb,),
        in_specs=[
            pl.BlockSpec((rb, wp), lambda i: (i, 0)),
            pl.BlockSpec((rb, wp), lambda i: (i, 0)),
            pl.BlockSpec((8, hid1), lambda i: (0, 0)),
            pl.BlockSpec((1, hid1), lambda i: (0, 0)),
            pl.BlockSpec((hid1, d_out), lambda i: (0, 0)),
            pl.BlockSpec((d_out, 2), lambda i: (0, 0)),
        ],
        out_specs=[
            pl.BlockSpec((rb, d_out + 16), lambda i: (i, 0)),
            pl.BlockSpec((rb, 16), lambda i: (i, 0)),
        ],
        out_shape=[
            jax.ShapeDtypeStruct((n, d_out + 16), jnp.float32),
            jax.ShapeDtypeStruct((n, 16), jnp.float32),
        ],
    )(p0, p1, bexp, b1, w2, a2v)


def _tc_final(q0, q1, b2, rb, n_out):
    d_out = b2.shape[1]
    wq = q0.shape[1]

    def body(q0_ref, q1_ref, b2_ref, out_ref):
        t = q0_ref[...] + q1_ref[...]
        num = t[:, :d_out]
        den = t[:, d_out:d_out + 1]
        out_ref[...] = num / (den + 1e-16) + b2_ref[...]

    return pl.pallas_call(
        body,
        grid=(n_out // rb,),
        in_specs=[
            pl.BlockSpec((rb, wq), lambda i: (i, 0)),
            pl.BlockSpec((rb, wq), lambda i: (i, 0)),
            pl.BlockSpec((1, d_out), lambda i: (0, 0)),
        ],
        out_specs=pl.BlockSpec((rb, d_out), lambda i: (i, 0)),
        out_shape=jax.ShapeDtypeStruct((n_out, d_out), jnp.float32),
    )(q0, q1, b2)


def kernel(x, adj, W1, a1_src, a1_dst, b1, W2, a2_src, a2_dst, b2):
    n, d_in = x.shape
    e = adj.shape[1]
    heads, hid = a1_src.shape
    hid1 = W1.shape[1]
    d_out = W2.shape[1]

    n_pad = ((n + _NS * 16 - 1) // (_NS * 16)) * (_NS * 16)
    nw = _NC * _NS
    k_chunks = (e + nw * _C - 1) // (nw * _C)
    e_pad = nw * k_chunks * _C

    karange = jnp.arange(hid1)
    a1s = jnp.zeros((hid1, heads), jnp.float32).at[
        karange, karange // hid].set(a1_src.reshape(hid1))
    a1d = jnp.zeros((hid1, heads), jnp.float32).at[
        karange, karange // hid].set(a1_dst.reshape(hid1))
    bexp = jnp.zeros((heads, hid1), jnp.float32).at[
        karange // hid, karange].set(1.0)
    a2v = jnp.concatenate([a2_src, a2_dst], axis=0).T
    b1r = b1.reshape(1, hid1)
    b2r = b2.reshape(1, d_out)

    srcp = jnp.concatenate([adj[0], jnp.zeros((e_pad - e,), jnp.int32)])
    dstp = jnp.concatenate([adj[1], jnp.full((e_pad - e,), n, jnp.int32)])

    z1 = jnp.zeros((n_pad, hid1 + 16), jnp.float32)
    z2 = jnp.zeros((n_pad, d_out + 16), jnp.float32)

    hs, ad = _tc_layer1(x, W1, a1s, a1d, rb=1000)
    ad = jnp.concatenate([ad, jnp.zeros((n_pad - n, 16), jnp.float32)])
    p = _sc_aggregate(hs, ad, srcp, dstp, z1,
                      n_pad=n_pad, k_chunks=k_chunks, dh=hid1, hdim=hid)

    hs2, ad2 = _tc_layer2(p[0], p[1], bexp, b1r, W2, a2v, rb=1000)
    ad2 = jnp.concatenate([ad2, jnp.zeros((n_pad - n, 16), jnp.float32)])
    q = _sc_aggregate(hs2, ad2, srcp, dstp, z2,
                      n_pad=n_pad, k_chunks=k_chunks, dh=d_out, hdim=d_out)

    return _tc_final(q[0], q[1], b2r, rb=1000, n_out=n)


# speedup vs baseline: 35.1341x; 35.1341x over previous
import functools

import jax
import jax.numpy as jnp
from jax import lax
from jax.experimental import pallas as pl
from jax.experimental.pallas import tpu as pltpu
from jax.experimental.pallas import tpu_sc as plsc

_NC = 2
_NS = 16
_L = 16
_C = 64

_SC_PARAMS = pltpu.CompilerParams(needs_layout_passes=False)


def _leaky_exp(z):
    return jnp.exp(jnp.where(z >= 0, z, 0.2 * z))


def _sc_gat(hs, lt, srcp, dstp, z128, *, n_acc, k_chunks, heads_mode):
    rpt = n_acc // _NS
    mesh = plsc.VectorSubcoreMesh(core_axis_name="c", subcore_axis_name="s")

    @functools.partial(
        pl.kernel,
        out_type=jax.ShapeDtypeStruct((_NC, n_acc, 128), jnp.float32),
        mesh=mesh,
        compiler_params=_SC_PARAMS,
        scratch_types=[
            pltpu.VMEM((_C,), jnp.int32),
            pltpu.VMEM((_C,), jnp.int32),
            pltpu.VMEM((_C, 128), jnp.float32),
            pltpu.VMEM((_C, 128), jnp.float32),
            pltpu.VMEM((_C, 128), jnp.float32),
            pltpu.VMEM_SHARED((n_acc, 128), jnp.float32),
            pltpu.SemaphoreType.DMA,
        ],
    )
    def body(hs_hbm, lt_hbm, src_hbm, dst_hbm, z_hbm, out_hbm,
             src_v, dst_v, hs_v, mw_v, ld_v, acc, sem):
        cid = lax.axis_index("c")
        sid = lax.axis_index("s")
        wid = sid * _NC + cid
        r0 = sid * rpt
        pltpu.sync_copy(z_hbm.at[pl.ds(r0, rpt)], acc.at[pl.ds(r0, rpt)])

        def zrow(i, c):
            for g in range(5, 8):
                mw_v[i, pl.ds(g * _L, _L)] = jnp.zeros((_L,), jnp.float32)
            return c

        lax.fori_loop(0, _C, zrow, 0)
        plsc.subcore_barrier()
        ebase = wid * (k_chunks * _C)

        def chunk(j, carry):
            off = ebase + j * _C
            pltpu.sync_copy(src_hbm.at[pl.ds(off, _C)], src_v)
            pltpu.sync_copy(dst_hbm.at[pl.ds(off, _C)], dst_v)
            pltpu.async_copy(hs_hbm.at[src_v], hs_v, sem).wait()
            pltpu.async_copy(lt_hbm.at[dst_v], ld_v, sem).wait()

            def blk16(j16, c1):
                for ii in range(_L):
                    e = j16 * _L + ii
                    vad = ld_v[e, pl.ds(0, _L)]
                    vas = hs_v[e, pl.ds(64, _L)]
                    w = _leaky_exp(vas + vad)
                    mw_v[e, pl.ds(64, _L)] = w
                    for g in range(4):
                        if heads_mode:
                            wb = jnp.where(
                                lax.iota(jnp.int32, _L) < 8,
                                jnp.full((_L,), w[2 * g], jnp.float32),
                                jnp.full((_L,), w[2 * g + 1], jnp.float32))
                        else:
                            wb = w
                        mw_v[e, pl.ds(g * _L, _L)] = (
                            hs_v[e, pl.ds(g * _L, _L)] * wb)
                return c1

            lax.fori_loop(0, _C // _L, blk16, 0)
            pltpu.sync_copy(mw_v, acc.at[dst_v], add=True)
            return carry

        lax.fori_loop(0, k_chunks, chunk, 0)
        plsc.subcore_barrier()
        pltpu.sync_copy(acc.at[pl.ds(r0, rpt)], out_hbm.at[cid, pl.ds(r0, rpt)])

    return body(hs, lt, srcp, dstp, z128)


def _tc_layer1(x, w1, a1s, a1d, rb):
    n, d_in = x.shape
    hid1 = w1.shape[1]
    heads = a1s.shape[1]

    def body(x_ref, w1_ref, a1s_ref, a1d_ref, hs_ref, ltd_ref):
        h = jnp.dot(x_ref[...], w1_ref[...], preferred_element_type=jnp.float32)
        als = jnp.dot(h, a1s_ref[...], preferred_element_type=jnp.float32)
        ald = jnp.dot(h, a1d_ref[...], preferred_element_type=jnp.float32)
        hs_ref[...] = jnp.concatenate(
            [h, als, jnp.zeros((rb, 128 - hid1 - heads), jnp.float32)], axis=1)
        ltd_ref[...] = jnp.concatenate(
            [ald, jnp.zeros((rb, 16 - heads), jnp.float32)], axis=1)

    return pl.pallas_call(
        body,
        grid=(n // rb,),
        in_specs=[
            pl.BlockSpec((rb, d_in), lambda i: (i, 0)),
            pl.BlockSpec((d_in, hid1), lambda i: (0, 0)),
            pl.BlockSpec((hid1, heads), lambda i: (0, 0)),
            pl.BlockSpec((hid1, heads), lambda i: (0, 0)),
        ],
        out_specs=[
            pl.BlockSpec((rb, 128), lambda i: (i, 0)),
            pl.BlockSpec((rb, 16), lambda i: (i, 0)),
        ],
        out_shape=[
            jax.ShapeDtypeStruct((n, 128), jnp.float32),
            jax.ShapeDtypeStruct((n, 16), jnp.float32),
        ],
    )(x, w1, a1s, a1d)


def _tc_layer2(p0, p1, bexp, b1, w2, a2v, rb):
    n = p0.shape[0]
    hid1 = b1.shape[1]
    d_out = w2.shape[1]

    def body(p0_ref, p1_ref, bexp_ref, b1_ref, w2_ref,
             a2v_ref, hs2_ref, ld2_ref):
        num = p0_ref[:, :hid1] + p1_ref[:, :hid1]
        den = p0_ref[:, hid1:hid1 + 8] + p1_ref[:, hid1:hid1 + 8]
        r = 1.0 / (den + 1e-16)
        rexp = jnp.dot(r, bexp_ref[...], preferred_element_type=jnp.float32)
        h1 = num * rexp + b1_ref[...]
        h1 = jnp.where(h1 > 0, h1, jnp.exp(jnp.minimum(h1, 0.0)) - 1.0)
        wa = jnp.dot(w2_ref[...], a2v_ref[...],
                     preferred_element_type=jnp.float32)
        al2 = jnp.dot(h1, wa, preferred_element_type=jnp.float32)
        ones16 = jnp.ones((1, 16), jnp.float32)
        hs2_ref[...] = jnp.concatenate(
            [h1, al2[:, :1] * ones16,
             jnp.zeros((rb, 128 - hid1 - 16), jnp.float32)], axis=1)
        ld2_ref[...] = al2[:, 1:2] * ones16

    return pl.pallas_call(
        body,
        grid=(n // rb,),
        in_specs=[
            pl.BlockSpec((rb, 128), lambda i: (i, 0)),
            pl.BlockSpec((rb, 128), lambda i: (i, 0)),
            pl.BlockSpec((8, hid1), lambda i: (0, 0)),
            pl.BlockSpec((1, hid1), lambda i: (0, 0)),
            pl.BlockSpec((hid1, d_out), lambda i: (0, 0)),
            pl.BlockSpec((d_out, 2), lambda i: (0, 0)),
        ],
        out_specs=[
            pl.BlockSpec((rb, 128), lambda i: (i, 0)),
            pl.BlockSpec((rb, 16), lambda i: (i, 0)),
        ],
        out_shape=[
            jax.ShapeDtypeStruct((n, 128), jnp.float32),
            jax.ShapeDtypeStruct((n, 16), jnp.float32),
        ],
    )(p0, p1, bexp, b1, w2, a2v)


def _tc_final(q0, q1, w2, b2, rb, n_out):
    hid1 = w2.shape[0]
    d_out = b2.shape[1]

    def body(q0_ref, q1_ref, w2_ref, b2_ref, out_ref):
        num = q0_ref[:, :hid1] + q1_ref[:, :hid1]
        den = q0_ref[:, hid1:hid1 + 1] + q1_ref[:, hid1:hid1 + 1]
        hbar = num / (den + 1e-16)
        out_ref[...] = jnp.dot(
            hbar, w2_ref[...], preferred_element_type=jnp.float32) + b2_ref[...]

    return pl.pallas_call(
        body,
        grid=(n_out // rb,),
        in_specs=[
            pl.BlockSpec((rb, 128), lambda i: (i, 0)),
            pl.BlockSpec((rb, 128), lambda i: (i, 0)),
            pl.BlockSpec((hid1, d_out), lambda i: (0, 0)),
            pl.BlockSpec((1, d_out), lambda i: (0, 0)),
        ],
        out_specs=pl.BlockSpec((rb, d_out), lambda i: (i, 0)),
        out_shape=jax.ShapeDtypeStruct((n_out, d_out), jnp.float32),
    )(q0, q1, w2, b2)


def kernel(x, adj, W1, a1_src, a1_dst, b1, W2, a2_src, a2_dst, b2):
    n, d_in = x.shape
    e = adj.shape[1]
    heads, hid = a1_src.shape
    hid1 = W1.shape[1]
    d_out = W2.shape[1]

    n_tab = ((n + _NS - 1) // _NS) * _NS + _NS
    n_acc = ((n + 127) // 128) * 128
    nw = _NC * _NS
    k_chunks = (e + nw * _C - 1) // (nw * _C)
    e_pad = nw * k_chunks * _C

    karange = jnp.arange(hid1)
    a1s = jnp.zeros((hid1, heads), jnp.float32).at[
        karange, karange // hid].set(a1_src.reshape(hid1))
    a1d = jnp.zeros((hid1, heads), jnp.float32).at[
        karange, karange // hid].set(a1_dst.reshape(hid1))
    bexp = jnp.zeros((heads, hid1), jnp.float32).at[
        karange // hid, karange].set(1.0)
    a2v = jnp.concatenate([a2_src, a2_dst], axis=0).T
    b1r = b1.reshape(1, hid1)
    b2r = b2.reshape(1, d_out)

    srcp = jnp.concatenate([adj[0], jnp.full((e_pad - e,), n, jnp.int32)])
    dstp = jnp.concatenate([adj[1], jnp.zeros((e_pad - e,), jnp.int32)])

    z128 = jnp.zeros((n_acc, 128), jnp.float32)

    hs, ltd = _tc_layer1(x, W1, a1s, a1d, rb=1000)
    sentinel = jnp.concatenate(
        [jnp.zeros((n_tab - n, hid1), jnp.float32),
         jnp.full((n_tab - n, 16), -1e30, jnp.float32),
         jnp.zeros((n_tab - n, 128 - hid1 - 16), jnp.float32)], axis=1)
    hs = jnp.concatenate([hs, sentinel])
    ltd = jnp.concatenate([ltd, jnp.zeros((n, 112), jnp.float32)], axis=1)
    ltd = jnp.concatenate([ltd, jnp.zeros((n_acc - n, 128), jnp.float32)])
    p = _sc_gat(hs, ltd, srcp, dstp, z128, n_acc=n_acc, k_chunks=k_chunks,
                heads_mode=True)

    hs2, ld2 = _tc_layer2(p[0, :n], p[1, :n], bexp, b1r, W2, a2v, rb=1000)
    hs2 = jnp.concatenate([hs2, sentinel])
    ld2 = jnp.concatenate([ld2, jnp.zeros((n, 112), jnp.float32)], axis=1)
    ld2 = jnp.concatenate([ld2, jnp.zeros((n_acc - n, 128), jnp.float32)])
    q = _sc_gat(hs2, ld2, srcp, dstp, z128, n_acc=n_acc, k_chunks=k_chunks,
                heads_mode=False)

    return _tc_final(q[0, :n], q[1, :n], W2, b2r, rb=1000, n_out=n)


# speedup vs baseline: 43.9881x; 1.2520x over previous
import functools

import jax
import jax.numpy as jnp
from jax import lax
from jax.experimental import pallas as pl
from jax.experimental.pallas import tpu as pltpu
from jax.experimental.pallas import tpu_sc as plsc

_NC = 2
_NS = 16
_L = 16
_C = 64

_SC_PARAMS = pltpu.CompilerParams(needs_layout_passes=False)


def _leaky_exp(z):
    return jnp.exp(jnp.where(z >= 0, z, 0.2 * z))


def _sc_gat(hs, lt, srcp, dstp, z128, *, n_acc, k_chunks, heads_mode):
    rpt = n_acc // _NS
    mesh = plsc.VectorSubcoreMesh(core_axis_name="c", subcore_axis_name="s")

    @functools.partial(
        pl.kernel,
        out_type=jax.ShapeDtypeStruct((_NC, n_acc, 128), jnp.float32),
        mesh=mesh,
        compiler_params=_SC_PARAMS,
        scratch_types=[
            pltpu.VMEM((_C,), jnp.int32),
            pltpu.VMEM((_C,), jnp.int32),
            pltpu.VMEM((_C, 128), jnp.float32),
            pltpu.VMEM((_C, 128), jnp.float32),
            pltpu.VMEM((_C, 128), jnp.float32),
            pltpu.VMEM_SHARED((n_acc, 128), jnp.float32),
            pltpu.SemaphoreType.DMA,
        ],
    )
    def body(hs_hbm, lt_hbm, src_hbm, dst_hbm, z_hbm, out_hbm,
             src_v, dst_v, hs_v, mw_v, ld_v, acc, sem):
        cid = lax.axis_index("c")
        sid = lax.axis_index("s")
        wid = sid * _NC + cid
        r0 = sid * rpt
        pltpu.sync_copy(z_hbm.at[pl.ds(r0, rpt)], acc.at[pl.ds(r0, rpt)])

        def zrow(i, c):
            for g in range(5, 8):
                mw_v[i, pl.ds(g * _L, _L)] = jnp.zeros((_L,), jnp.float32)
            return c

        lax.fori_loop(0, _C, zrow, 0)
        plsc.subcore_barrier()
        ebase = wid * (k_chunks * _C)

        def chunk(j, carry):
            off = ebase + j * _C
            pltpu.sync_copy(src_hbm.at[pl.ds(off, _C)], src_v)
            pltpu.sync_copy(dst_hbm.at[pl.ds(off, _C)], dst_v)
            cp_h = pltpu.async_copy(hs_hbm.at[src_v], hs_v, sem)
            cp_l = pltpu.async_copy(lt_hbm.at[dst_v], ld_v, sem)
            cp_h.wait()
            cp_l.wait()

            def blk16(j16, c1):
                for ii in range(_L):
                    e = j16 * _L + ii
                    vad = ld_v[e, pl.ds(0, _L)]
                    vas = hs_v[e, pl.ds(64, _L)]
                    w = _leaky_exp(vas + vad)
                    mw_v[e, pl.ds(64, _L)] = w
                    for g in range(4):
                        if heads_mode:
                            wb = jnp.where(
                                lax.iota(jnp.int32, _L) < 8,
                                jnp.full((_L,), w[2 * g], jnp.float32),
                                jnp.full((_L,), w[2 * g + 1], jnp.float32))
                        else:
                            wb = w
                        mw_v[e, pl.ds(g * _L, _L)] = (
                            hs_v[e, pl.ds(g * _L, _L)] * wb)
                return c1

            lax.fori_loop(0, _C // _L, blk16, 0)
            pltpu.sync_copy(mw_v, acc.at[dst_v], add=True)
            return carry

        lax.fori_loop(0, k_chunks, chunk, 0)
        plsc.subcore_barrier()
        pltpu.sync_copy(acc.at[pl.ds(r0, rpt)], out_hbm.at[cid, pl.ds(r0, rpt)])

    return body(hs, lt, srcp, dstp, z128)


def _tc_layer1(x, w1, a1s, a1d, rb):
    n, d_in = x.shape
    hid1 = w1.shape[1]
    heads = a1s.shape[1]

    def body(x_ref, w1_ref, a1s_ref, a1d_ref, hs_ref, ltd_ref):
        h = jnp.dot(x_ref[...], w1_ref[...], preferred_element_type=jnp.float32)
        als = jnp.dot(h, a1s_ref[...], preferred_element_type=jnp.float32)
        ald = jnp.dot(h, a1d_ref[...], preferred_element_type=jnp.float32)
        hs_ref[...] = jnp.concatenate(
            [h, als, jnp.zeros((rb, 128 - hid1 - heads), jnp.float32)], axis=1)
        ltd_ref[...] = jnp.concatenate(
            [ald, jnp.zeros((rb, 16 - heads), jnp.float32)], axis=1)

    return pl.pallas_call(
        body,
        grid=(n // rb,),
        in_specs=[
            pl.BlockSpec((rb, d_in), lambda i: (i, 0)),
            pl.BlockSpec((d_in, hid1), lambda i: (0, 0)),
            pl.BlockSpec((hid1, heads), lambda i: (0, 0)),
            pl.BlockSpec((hid1, heads), lambda i: (0, 0)),
        ],
        out_specs=[
            pl.BlockSpec((rb, 128), lambda i: (i, 0)),
            pl.BlockSpec((rb, 16), lambda i: (i, 0)),
        ],
        out_shape=[
            jax.ShapeDtypeStruct((n, 128), jnp.float32),
            jax.ShapeDtypeStruct((n, 16), jnp.float32),
        ],
    )(x, w1, a1s, a1d)


def _tc_layer2(p0, p1, bexp, b1, w2, a2v, rb):
    n = p0.shape[0]
    hid1 = b1.shape[1]
    d_out = w2.shape[1]

    def body(p0_ref, p1_ref, bexp_ref, b1_ref, w2_ref,
             a2v_ref, hs2_ref, ld2_ref):
        num = p0_ref[:, :hid1] + p1_ref[:, :hid1]
        den = p0_ref[:, hid1:hid1 + 8] + p1_ref[:, hid1:hid1 + 8]
        r = 1.0 / (den + 1e-16)
        rexp = jnp.dot(r, bexp_ref[...], preferred_element_type=jnp.float32)
        h1 = num * rexp + b1_ref[...]
        h1 = jnp.where(h1 > 0, h1, jnp.exp(jnp.minimum(h1, 0.0)) - 1.0)
        wa = jnp.dot(w2_ref[...], a2v_ref[...],
                     preferred_element_type=jnp.float32)
        al2 = jnp.dot(h1, wa, preferred_element_type=jnp.float32)
        ones16 = jnp.ones((1, 16), jnp.float32)
        hs2_ref[...] = jnp.concatenate(
            [h1, al2[:, :1] * ones16,
             jnp.zeros((rb, 128 - hid1 - 16), jnp.float32)], axis=1)
        ld2_ref[...] = al2[:, 1:2] * ones16

    return pl.pallas_call(
        body,
        grid=(n // rb,),
        in_specs=[
            pl.BlockSpec((rb, 128), lambda i: (i, 0)),
            pl.BlockSpec((rb, 128), lambda i: (i, 0)),
            pl.BlockSpec((8, hid1), lambda i: (0, 0)),
            pl.BlockSpec((1, hid1), lambda i: (0, 0)),
            pl.BlockSpec((hid1, d_out), lambda i: (0, 0)),
            pl.BlockSpec((d_out, 2), lambda i: (0, 0)),
        ],
        out_specs=[
            pl.BlockSpec((rb, 128), lambda i: (i, 0)),
            pl.BlockSpec((rb, 16), lambda i: (i, 0)),
        ],
        out_shape=[
            jax.ShapeDtypeStruct((n, 128), jnp.float32),
            jax.ShapeDtypeStruct((n, 16), jnp.float32),
        ],
    )(p0, p1, bexp, b1, w2, a2v)


def _tc_final(q0, q1, w2, b2, rb, n_out):
    hid1 = w2.shape[0]
    d_out = b2.shape[1]

    def body(q0_ref, q1_ref, w2_ref, b2_ref, out_ref):
        num = q0_ref[:, :hid1] + q1_ref[:, :hid1]
        den = q0_ref[:, hid1:hid1 + 1] + q1_ref[:, hid1:hid1 + 1]
        hbar = num / (den + 1e-16)
        out_ref[...] = jnp.dot(
            hbar, w2_ref[...], preferred_element_type=jnp.float32) + b2_ref[...]

    return pl.pallas_call(
        body,
        grid=(n_out // rb,),
        in_specs=[
            pl.BlockSpec((rb, 128), lambda i: (i, 0)),
            pl.BlockSpec((rb, 128), lambda i: (i, 0)),
            pl.BlockSpec((hid1, d_out), lambda i: (0, 0)),
            pl.BlockSpec((1, d_out), lambda i: (0, 0)),
        ],
        out_specs=pl.BlockSpec((rb, d_out), lambda i: (i, 0)),
        out_shape=jax.ShapeDtypeStruct((n_out, d_out), jnp.float32),
    )(q0, q1, w2, b2)


def kernel(x, adj, W1, a1_src, a1_dst, b1, W2, a2_src, a2_dst, b2):
    n, d_in = x.shape
    e = adj.shape[1]
    heads, hid = a1_src.shape
    hid1 = W1.shape[1]
    d_out = W2.shape[1]

    n_tab = ((n + _NS - 1) // _NS) * _NS + _NS
    n_acc = ((n + 127) // 128) * 128
    nw = _NC * _NS
    k_chunks = (e + nw * _C - 1) // (nw * _C)
    e_pad = nw * k_chunks * _C

    karange = jnp.arange(hid1)
    a1s = jnp.zeros((hid1, heads), jnp.float32).at[
        karange, karange // hid].set(a1_src.reshape(hid1))
    a1d = jnp.zeros((hid1, heads), jnp.float32).at[
        karange, karange // hid].set(a1_dst.reshape(hid1))
    bexp = jnp.zeros((heads, hid1), jnp.float32).at[
        karange // hid, karange].set(1.0)
    a2v = jnp.concatenate([a2_src, a2_dst], axis=0).T
    b1r = b1.reshape(1, hid1)
    b2r = b2.reshape(1, d_out)

    srcp = jnp.concatenate([adj[0], jnp.full((e_pad - e,), n, jnp.int32)])
    dstp = jnp.concatenate([adj[1], jnp.zeros((e_pad - e,), jnp.int32)])

    z128 = jnp.zeros((n_acc, 128), jnp.float32)

    hs, ltd = _tc_layer1(x, W1, a1s, a1d, rb=1000)
    sentinel = jnp.concatenate(
        [jnp.zeros((n_tab - n, hid1), jnp.float32),
         jnp.full((n_tab - n, 16), -1e30, jnp.float32),
         jnp.zeros((n_tab - n, 128 - hid1 - 16), jnp.float32)], axis=1)
    hs = jnp.concatenate([hs, sentinel])
    ltd = jnp.concatenate([ltd, jnp.zeros((n, 112), jnp.float32)], axis=1)
    ltd = jnp.concatenate([ltd, jnp.zeros((n_acc - n, 128), jnp.float32)])
    p = _sc_gat(hs, ltd, srcp, dstp, z128, n_acc=n_acc, k_chunks=k_chunks,
                heads_mode=True)

    hs2, ld2 = _tc_layer2(p[0, :n], p[1, :n], bexp, b1r, W2, a2v, rb=1000)
    hs2 = jnp.concatenate([hs2, sentinel])
    ld2 = jnp.concatenate([ld2, jnp.zeros((n, 112), jnp.float32)], axis=1)
    ld2 = jnp.concatenate([ld2, jnp.zeros((n_acc - n, 128), jnp.float32)])
    q = _sc_gat(hs2, ld2, srcp, dstp, z128, n_acc=n_acc, k_chunks=k_chunks,
                heads_mode=False)

    return _tc_final(q[0, :n], q[1, :n], W2, b2r, rb=1000, n_out=n)
